# baseline (speedup 1.0000x reference)
import functools

import jax
import jax.numpy as jnp
from jax import lax
from jax.experimental import pallas as pl
from jax.experimental.pallas import tpu as pltpu
from jax.experimental.pallas import tpu_sc as plsc

_NC = 2
_NS = 16
_NW = _NC * _NS
_L = 16
_B = 16
_R = 3
_NH = 1
_NCHUNK = 8


def _combine(inp, out, offs16, spl16, *, in_len, out_len, d):
    rows_per_w = in_len // _NW
    nbatch = rows_per_w // _B
    h_w = d // _NH
    assert nbatch % _R == 2 and nbatch >= 8 and d % (_NH * 128) == 0

    mesh = plsc.VectorSubcoreMesh(core_axis_name="c", subcore_axis_name="s")

    @functools.partial(
        pl.kernel,
        mesh=mesh,
        out_type=jax.ShapeDtypeStruct((out_len, d), jnp.float32),
        scratch_types=[
            pltpu.VMEM((_L,), jnp.int32),
            pltpu.VMEM((_L,), jnp.int32),
        ]
        + [pltpu.VMEM((_B, h_w), jnp.float32) for _ in range(_NH * _R)]
        + [pltpu.SemaphoreType.DMA for _ in range(2 * _NH * _R)],
    )
    def k(inp_hbm, out_hbm, offs_hbm, spl_hbm, o2_hbm, offs_v, spl_v, *ring):
        wid = lax.axis_index("s") * _NC + lax.axis_index("c")
        iota = lax.iota(jnp.int32, _L)
        nslot = _NH * _R
        rows = [ring[h * _R:(h + 1) * _R] for h in range(_NH)]
        gsem = [ring[nslot + h * _R:nslot + (h + 1) * _R] for h in range(_NH)]
        ssem = [ring[2 * nslot + h * _R:2 * nslot + (h + 1) * _R]
                for h in range(_NH)]

        pltpu.sync_copy(offs_hbm, offs_v)
        pltpu.sync_copy(spl_hbm, spl_v)
        offs = offs_v[...]
        spl = spl_v[...]
        in_off = [offs[c] for c in range(_NCHUNK)]
        out_off = [offs[c + 8] for c in range(_NCHUNK)]
        splits = [spl[c] for c in range(_NCHUNK)]
        shift = [out_off[c] - in_off[c] for c in range(_NCHUNK)]
        end = [out_off[c] + splits[c] for c in range(_NCHUNK)]
        gpre, acc = [], jnp.int32(0)
        for c in range(_NCHUNK):
            gpre.append(acc)
            nxt = out_off[c + 1] if c + 1 < _NCHUNK else jnp.int32(out_len)
            acc = acc + (nxt - end[c])

        base = wid * rows_per_w

        def dst_rows(i):
            r = pl.multiple_of(base + i * _B, _B) + iota
            cnt = jnp.zeros((_L,), jnp.int32)
            for c in range(1, _NCHUNK):
                cnt = cnt + jnp.where(r >= in_off[c], 1, 0)
            sh = jnp.zeros((_L,), jnp.int32)
            for c in range(1, _NCHUNK):
                sh = jnp.where(cnt == c, shift[c], sh)
            return r + sh

        def g_copy(i, h, s):
            r0 = pl.multiple_of(base + i * _B, _B)
            src = inp_hbm.at[pl.ds(r0, _B), pl.ds(h * h_w, h_w)]
            return pltpu.make_async_copy(src, rows[h][s], gsem[h][s])

        def s_copy(i, h, s):
            dst = o2_hbm.at[dst_rows(i), pl.ds(h * h_w, h_w)]
            return pltpu.make_async_copy(rows[h][s], dst, ssem[h][s])

        def step(i, *, head, tail):
            s_cur = i % _R
            s_old = (i + 1) % _R
            for h in range(_NH):
                if not head:
                    s_copy(i - 2, h, s_old).wait()
                if not tail:
                    g_copy(i + 1, h, s_old).start()
            for h in range(_NH):
                g_copy(i, h, s_cur).wait()
                s_copy(i, h, s_cur).start()

        for h in range(_NH):
            g_copy(0, h, 0).start()
        step(0, head=True, tail=False)
        step(1, head=True, tail=False)
        step(2, head=False, tail=False)

        def group(sg, carry):
            for kk in range(_R):
                i = _R * sg + kk
                s_cur = kk
                s_old = (kk + 1) % _R
                for h in range(_NH):
                    s_copy(i - 2, h, s_old).wait()
                    g_copy(i + 1, h, s_old).start()
                for h in range(_NH):
                    g_copy(i, h, s_cur).wait()
                    s_copy(i, h, s_cur).start()
            return carry

        lax.fori_loop(1, nbatch // _R, group, 0)

        step(nbatch - 2, head=False, tail=False)
        step(nbatch - 1, head=False, tail=True)
        for i in (nbatch - 2, nbatch - 1):
            for h in range(_NH):
                s_copy(i, h, i % _R).wait()

        @pl.when(wid < (out_len - in_len) // _L)
        def _():
            p = wid * _L + iota
            cnt = jnp.zeros((_L,), jnp.int32)
            for c in range(1, _NCHUNK):
                cnt = cnt + jnp.where(p >= gpre[c], 1, 0)
            rb = jnp.full((_L,), end[0] - gpre[0], jnp.int32)
            for c in range(1, _NCHUNK):
                rb = jnp.where(cnt == c, end[c] - gpre[c], rb)
            prow = rb + p
            for h in range(_NH):
                src = out_hbm.at[prow, pl.ds(h * h_w, h_w)]
                pltpu.make_async_copy(src, rows[h][0], gsem[h][0]).start()
            for h in range(_NH):
                pltpu.make_async_copy(
                    out_hbm.at[prow, pl.ds(h * h_w, h_w)],
                    rows[h][0], gsem[h][0]).wait()
                dst = o2_hbm.at[prow, pl.ds(h * h_w, h_w)]
                pltpu.make_async_copy(rows[h][0], dst, ssem[h][0]).start()
            for h in range(_NH):
                dst = o2_hbm.at[prow, pl.ds(h * h_w, h_w)]
                pltpu.make_async_copy(rows[h][0], dst, ssem[h][0]).wait()

    return k(inp, out, offs16, spl16)


def kernel(inp, out, in_splits_offsets, out_splits_offsets):
    iso = in_splits_offsets.astype(jnp.int32)
    oso = out_splits_offsets.astype(jnp.int32)
    offs16 = jnp.concatenate([iso[1], oso[1]])
    spl16 = jnp.concatenate([iso[0], jnp.zeros((8,), jnp.int32)])
    return _combine(inp, out, offs16, spl16,
                    in_len=inp.shape[0], out_len=out.shape[0],
                    d=inp.shape[1])


# speedup vs baseline: 1.0084x; 1.0084x over previous
import functools

import jax
import jax.numpy as jnp
from jax import lax
from jax.experimental import pallas as pl
from jax.experimental.pallas import tpu as pltpu
from jax.experimental.pallas import tpu_sc as plsc

_NC = 2
_NS = 16
_NW = _NC * _NS
_L = 16
_B = 16
_R = 3
_NH = 4
_NCHUNK = 8


def _combine(inp, out, offs16, spl16, *, in_len, out_len, d):
    rows_per_w = in_len // _NW
    nbatch = rows_per_w // _B
    h_w = d // _NH
    assert nbatch % _R == 2 and nbatch >= 8 and d % (_NH * 128) == 0

    mesh = plsc.VectorSubcoreMesh(core_axis_name="c", subcore_axis_name="s")

    @functools.partial(
        pl.kernel,
        mesh=mesh,
        out_type=jax.ShapeDtypeStruct((out_len, d), jnp.float32),
        scratch_types=[
            pltpu.VMEM((_L,), jnp.int32),
            pltpu.VMEM((_L,), jnp.int32),
        ]
        + [pltpu.VMEM((_B, h_w), jnp.float32) for _ in range(_NH * _R)]
        + [pltpu.SemaphoreType.DMA for _ in range(2 * _NH * _R)],
    )
    def k(inp_hbm, out_hbm, offs_hbm, spl_hbm, o2_hbm, offs_v, spl_v, *ring):
        wid = lax.axis_index("s") * _NC + lax.axis_index("c")
        iota = lax.iota(jnp.int32, _L)
        nslot = _NH * _R
        rows = [ring[h * _R:(h + 1) * _R] for h in range(_NH)]
        gsem = [ring[nslot + h * _R:nslot + (h + 1) * _R] for h in range(_NH)]
        ssem = [ring[2 * nslot + h * _R:2 * nslot + (h + 1) * _R]
                for h in range(_NH)]

        pltpu.sync_copy(offs_hbm, offs_v)
        pltpu.sync_copy(spl_hbm, spl_v)
        offs = offs_v[...]
        spl = spl_v[...]
        in_off = [offs[c] for c in range(_NCHUNK)]
        out_off = [offs[c + 8] for c in range(_NCHUNK)]
        splits = [spl[c] for c in range(_NCHUNK)]
        shift = [out_off[c] - in_off[c] for c in range(_NCHUNK)]
        end = [out_off[c] + splits[c] for c in range(_NCHUNK)]
        gpre, acc = [], jnp.int32(0)
        for c in range(_NCHUNK):
            gpre.append(acc)
            nxt = out_off[c + 1] if c + 1 < _NCHUNK else jnp.int32(out_len)
            acc = acc + (nxt - end[c])

        base = wid * rows_per_w

        def dst_rows(i):
            r = pl.multiple_of(base + i * _B, _B) + iota
            cnt = jnp.zeros((_L,), jnp.int32)
            for c in range(1, _NCHUNK):
                cnt = cnt + jnp.where(r >= in_off[c], 1, 0)
            sh = jnp.zeros((_L,), jnp.int32)
            for c in range(1, _NCHUNK):
                sh = jnp.where(cnt == c, shift[c], sh)
            return r + sh

        def g_copy(i, h, s):
            r0 = pl.multiple_of(base + i * _B, _B)
            src = inp_hbm.at[pl.ds(r0, _B), pl.ds(h * h_w, h_w)]
            return pltpu.make_async_copy(src, rows[h][s], gsem[h][s])

        def s_copy(i, h, s):
            dst = o2_hbm.at[dst_rows(i), pl.ds(h * h_w, h_w)]
            return pltpu.make_async_copy(rows[h][s], dst, ssem[h][s])

        def step(i, *, head, tail):
            s_cur = i % _R
            s_old = (i + 1) % _R
            for h in range(_NH):
                if not head:
                    s_copy(i - 2, h, s_old).wait()
                if not tail:
                    g_copy(i + 1, h, s_old).start()
            for h in range(_NH):
                g_copy(i, h, s_cur).wait()
                s_copy(i, h, s_cur).start()

        for h in range(_NH):
            g_copy(0, h, 0).start()
        step(0, head=True, tail=False)
        step(1, head=True, tail=False)
        step(2, head=False, tail=False)

        def group(sg, carry):
            for kk in range(_R):
                i = _R * sg + kk
                s_cur = kk
                s_old = (kk + 1) % _R
                for h in range(_NH):
                    s_copy(i - 2, h, s_old).wait()
                    g_copy(i + 1, h, s_old).start()
                for h in range(_NH):
                    g_copy(i, h, s_cur).wait()
                    s_copy(i, h, s_cur).start()
            return carry

        lax.fori_loop(1, nbatch // _R, group, 0)

        step(nbatch - 2, head=False, tail=False)
        step(nbatch - 1, head=False, tail=True)
        for i in (nbatch - 2, nbatch - 1):
            for h in range(_NH):
                s_copy(i, h, i % _R).wait()

        @pl.when(wid < (out_len - in_len) // _L)
        def _():
            p = wid * _L + iota
            cnt = jnp.zeros((_L,), jnp.int32)
            for c in range(1, _NCHUNK):
                cnt = cnt + jnp.where(p >= gpre[c], 1, 0)
            rb = jnp.full((_L,), end[0] - gpre[0], jnp.int32)
            for c in range(1, _NCHUNK):
                rb = jnp.where(cnt == c, end[c] - gpre[c], rb)
            prow = rb + p
            for h in range(_NH):
                src = out_hbm.at[prow, pl.ds(h * h_w, h_w)]
                pltpu.make_async_copy(src, rows[h][0], gsem[h][0]).start()
            for h in range(_NH):
                pltpu.make_async_copy(
                    out_hbm.at[prow, pl.ds(h * h_w, h_w)],
                    rows[h][0], gsem[h][0]).wait()
                dst = o2_hbm.at[prow, pl.ds(h * h_w, h_w)]
                pltpu.make_async_copy(rows[h][0], dst, ssem[h][0]).start()
            for h in range(_NH):
                dst = o2_hbm.at[prow, pl.ds(h * h_w, h_w)]
                pltpu.make_async_copy(rows[h][0], dst, ssem[h][0]).wait()

    return k(inp, out, offs16, spl16)


def kernel(inp, out, in_splits_offsets, out_splits_offsets):
    iso = in_splits_offsets.astype(jnp.int32)
    oso = out_splits_offsets.astype(jnp.int32)
    offs16 = jnp.concatenate([iso[1], oso[1]])
    spl16 = jnp.concatenate([iso[0], jnp.zeros((8,), jnp.int32)])
    return _combine(inp, out, offs16, spl16,
                    in_len=inp.shape[0], out_len=out.shape[0],
                    d=inp.shape[1])


# speedup vs baseline: 1.0092x; 1.0008x over previous
import functools

import jax
import jax.numpy as jnp
from jax import lax
from jax.experimental import pallas as pl
from jax.experimental.pallas import tpu as pltpu
from jax.experimental.pallas import tpu_sc as plsc

_NC = 2
_NS = 16
_NW = _NC * _NS
_L = 16
_B = 16
_R = 3
_NH = 2
_NCHUNK = 8


def _combine(inp, out, offs16, spl16, *, in_len, out_len, d):
    rows_per_w = in_len // _NW
    nbatch = rows_per_w // _B
    h_w = d // _NH
    assert nbatch % _R == 2 and nbatch >= 8 and d % (_NH * 128) == 0

    mesh = plsc.VectorSubcoreMesh(core_axis_name="c", subcore_axis_name="s")

    @functools.partial(
        pl.kernel,
        mesh=mesh,
        out_type=jax.ShapeDtypeStruct((out_len, d), jnp.float32),
        scratch_types=[
            pltpu.VMEM((_L,), jnp.int32),
            pltpu.VMEM((_L,), jnp.int32),
        ]
        + [pltpu.VMEM((_B, h_w), jnp.float32) for _ in range(_NH * _R)]
        + [pltpu.SemaphoreType.DMA for _ in range(2 * _NH * _R)],
    )
    def k(inp_hbm, out_hbm, offs_hbm, spl_hbm, o2_hbm, offs_v, spl_v, *ring):
        wid = lax.axis_index("s") * _NC + lax.axis_index("c")
        iota = lax.iota(jnp.int32, _L)
        nslot = _NH * _R
        rows = [ring[h * _R:(h + 1) * _R] for h in range(_NH)]
        gsem = [ring[nslot + h * _R:nslot + (h + 1) * _R] for h in range(_NH)]
        ssem = [ring[2 * nslot + h * _R:2 * nslot + (h + 1) * _R]
                for h in range(_NH)]

        pltpu.sync_copy(offs_hbm, offs_v)
        pltpu.sync_copy(spl_hbm, spl_v)
        offs = offs_v[...]
        spl = spl_v[...]
        in_off = [offs[c] for c in range(_NCHUNK)]
        out_off = [offs[c + 8] for c in range(_NCHUNK)]
        splits = [spl[c] for c in range(_NCHUNK)]
        shift = [out_off[c] - in_off[c] for c in range(_NCHUNK)]
        end = [out_off[c] + splits[c] for c in range(_NCHUNK)]
        gpre, acc = [], jnp.int32(0)
        for c in range(_NCHUNK):
            gpre.append(acc)
            nxt = out_off[c + 1] if c + 1 < _NCHUNK else jnp.int32(out_len)
            acc = acc + (nxt - end[c])

        base = wid * rows_per_w

        def dst_rows(i):
            r = pl.multiple_of(base + i * _B, _B) + iota
            cnt = jnp.zeros((_L,), jnp.int32)
            for c in range(1, _NCHUNK):
                cnt = cnt + jnp.where(r >= in_off[c], 1, 0)
            sh = jnp.zeros((_L,), jnp.int32)
            for c in range(1, _NCHUNK):
                sh = jnp.where(cnt == c, shift[c], sh)
            return r + sh

        def g_copy(i, h, s):
            r0 = pl.multiple_of(base + i * _B, _B)
            src = inp_hbm.at[pl.ds(r0, _B), pl.ds(h * h_w, h_w)]
            return pltpu.make_async_copy(src, rows[h][s], gsem[h][s])

        def s_copy(i, h, s):
            dst = o2_hbm.at[dst_rows(i), pl.ds(h * h_w, h_w)]
            return pltpu.make_async_copy(rows[h][s], dst, ssem[h][s])

        def step(i, *, head, tail):
            s_cur = i % _R
            s_old = (i + 1) % _R
            for h in range(_NH):
                if not head:
                    s_copy(i - 2, h, s_old).wait()
                if not tail:
                    g_copy(i + 1, h, s_old).start()
            for h in range(_NH):
                g_copy(i, h, s_cur).wait()
                s_copy(i, h, s_cur).start()

        for h in range(_NH):
            g_copy(0, h, 0).start()
        step(0, head=True, tail=False)
        step(1, head=True, tail=False)
        step(2, head=False, tail=False)

        def group(sg, carry):
            for kk in range(_R):
                i = _R * sg + kk
                s_cur = kk
                s_old = (kk + 1) % _R
                for h in range(_NH):
                    s_copy(i - 2, h, s_old).wait()
                    g_copy(i + 1, h, s_old).start()
                for h in range(_NH):
                    g_copy(i, h, s_cur).wait()
                    s_copy(i, h, s_cur).start()
            return carry

        lax.fori_loop(1, nbatch // _R, group, 0)

        step(nbatch - 2, head=False, tail=False)
        step(nbatch - 1, head=False, tail=True)
        for i in (nbatch - 2, nbatch - 1):
            for h in range(_NH):
                s_copy(i, h, i % _R).wait()

        @pl.when(wid < (out_len - in_len) // _L)
        def _():
            p = wid * _L + iota
            cnt = jnp.zeros((_L,), jnp.int32)
            for c in range(1, _NCHUNK):
                cnt = cnt + jnp.where(p >= gpre[c], 1, 0)
            rb = jnp.full((_L,), end[0] - gpre[0], jnp.int32)
            for c in range(1, _NCHUNK):
                rb = jnp.where(cnt == c, end[c] - gpre[c], rb)
            prow = rb + p
            for h in range(_NH):
                src = out_hbm.at[prow, pl.ds(h * h_w, h_w)]
                pltpu.make_async_copy(src, rows[h][0], gsem[h][0]).start()
            for h in range(_NH):
                pltpu.make_async_copy(
                    out_hbm.at[prow, pl.ds(h * h_w, h_w)],
                    rows[h][0], gsem[h][0]).wait()
                dst = o2_hbm.at[prow, pl.ds(h * h_w, h_w)]
                pltpu.make_async_copy(rows[h][0], dst, ssem[h][0]).start()
            for h in range(_NH):
                dst = o2_hbm.at[prow, pl.ds(h * h_w, h_w)]
                pltpu.make_async_copy(rows[h][0], dst, ssem[h][0]).wait()

    return k(inp, out, offs16, spl16)


def kernel(inp, out, in_splits_offsets, out_splits_offsets):
    iso = in_splits_offsets.astype(jnp.int32)
    oso = out_splits_offsets.astype(jnp.int32)
    offs16 = jnp.concatenate([iso[1], oso[1]])
    spl16 = jnp.concatenate([iso[0], jnp.zeros((8,), jnp.int32)])
    return _combine(inp, out, offs16, spl16,
                    in_len=inp.shape[0], out_len=out.shape[0],
                    d=inp.shape[1])


# speedup vs baseline: 1.0166x; 1.0074x over previous
import functools

import jax
import jax.numpy as jnp
from jax import lax
from jax.experimental import pallas as pl
from jax.experimental.pallas import tpu as pltpu
from jax.experimental.pallas import tpu_sc as plsc

_NC = 2
_NS = 16
_NW = _NC * _NS
_L = 16
_B = 16
_R = 3
_NH = 2
_NCHUNK = 8


def _combine(inp, out, offs16, spl16, *, in_len, out_len, d):
    rows_per_w = in_len // _NW
    nbatch = rows_per_w // _B
    h_w = d // _NH
    assert nbatch % _R == 2 and nbatch >= 8 and d % (_NH * 128) == 0

    mesh = plsc.VectorSubcoreMesh(core_axis_name="c", subcore_axis_name="s")

    @functools.partial(
        pl.kernel,
        mesh=mesh,
        out_type=jax.ShapeDtypeStruct((out_len, d), jnp.float32),
        scratch_types=[
            pltpu.VMEM((_L,), jnp.int32),
            pltpu.VMEM((_L,), jnp.int32),
        ]
        + [pltpu.VMEM((_B, h_w), jnp.float32) for _ in range(_NH * _R)]
        + [pltpu.SemaphoreType.DMA for _ in range(2 * _NH * _R)],
    )
    def k(inp_hbm, out_hbm, offs_hbm, spl_hbm, o2_hbm, offs_v, spl_v, *ring):
        wid = lax.axis_index("s") * _NC + lax.axis_index("c")
        iota = lax.iota(jnp.int32, _L)
        nslot = _NH * _R
        rows = [ring[h * _R:(h + 1) * _R] for h in range(_NH)]
        gsem = [ring[nslot + h * _R:nslot + (h + 1) * _R] for h in range(_NH)]
        ssem = [ring[2 * nslot + h * _R:2 * nslot + (h + 1) * _R]
                for h in range(_NH)]

        pltpu.sync_copy(offs_hbm, offs_v)
        pltpu.sync_copy(spl_hbm, spl_v)
        offs = offs_v[...]
        spl = spl_v[...]
        in_off = [offs[c] for c in range(_NCHUNK)]
        out_off = [offs[c + 8] for c in range(_NCHUNK)]
        splits = [spl[c] for c in range(_NCHUNK)]
        shift = [out_off[c] - in_off[c] for c in range(_NCHUNK)]
        end = [out_off[c] + splits[c] for c in range(_NCHUNK)]
        gpre, acc = [], jnp.int32(0)
        for c in range(_NCHUNK):
            gpre.append(acc)
            nxt = out_off[c + 1] if c + 1 < _NCHUNK else jnp.int32(out_len)
            acc = acc + (nxt - end[c])

        base = wid * rows_per_w

        def dst_rows(i):
            r = pl.multiple_of(base + i * _B, _B) + iota
            cnt = jnp.zeros((_L,), jnp.int32)
            for c in range(1, _NCHUNK):
                cnt = cnt + jnp.where(r >= in_off[c], 1, 0)
            sh = jnp.zeros((_L,), jnp.int32)
            for c in range(1, _NCHUNK):
                sh = jnp.where(cnt == c, shift[c], sh)
            return r + sh

        def g_copy(i, h, s):
            r0 = pl.multiple_of(base + i * _B, _B)
            src = inp_hbm.at[pl.ds(r0, _B), pl.ds(h * h_w, h_w)]
            return pltpu.make_async_copy(src, rows[h][s], gsem[h][s])

        def s_copy(i, h, s):
            dst = o2_hbm.at[dst_rows(i), pl.ds(h * h_w, h_w)]
            return pltpu.make_async_copy(rows[h][s], dst, ssem[h][s])

        def step(i, *, head, tail):
            s_cur = i % _R
            s_old = (i + 1) % _R
            for h in range(_NH):
                if not head:
                    s_copy(i - 2, h, s_old).wait()
                if not tail:
                    g_copy(i + 1, h, s_old).start()
            for h in range(_NH):
                g_copy(i, h, s_cur).wait()
                s_copy(i, h, s_cur).start()

        for h in range(_NH):
            g_copy(0, h, 0).start()
        step(0, head=True, tail=False)
        step(1, head=True, tail=False)
        step(2, head=False, tail=False)

        def group(sg, carry):
            for kk in range(_R):
                i = _R * sg + kk
                s_cur = kk
                s_old = (kk + 1) % _R
                for h in range(_NH):
                    s_copy(i - 2, h, s_old).wait()
                    g_copy(i + 1, h, s_old).start()
                for h in range(_NH):
                    g_copy(i, h, s_cur).wait()
                    s_copy(i, h, s_cur).start()
            return carry

        lax.fori_loop(1, nbatch // _R, group, 0)

        step(nbatch - 2, head=False, tail=False)
        step(nbatch - 1, head=False, tail=True)

        has_pad = wid < (out_len - in_len) // _L
        p = wid * _L + iota
        cnt = jnp.zeros((_L,), jnp.int32)
        for c in range(1, _NCHUNK):
            cnt = cnt + jnp.where(p >= gpre[c], 1, 0)
        rb = jnp.full((_L,), end[0] - gpre[0], jnp.int32)
        for c in range(1, _NCHUNK):
            rb = jnp.where(cnt == c, end[c] - gpre[c], rb)
        prow = rb + p
        ps = (nbatch - 3) % _R

        @pl.when(has_pad)
        def _():
            for h in range(_NH):
                src = out_hbm.at[prow, pl.ds(h * h_w, h_w)]
                pltpu.make_async_copy(src, rows[h][ps], gsem[h][ps]).start()

        for i in (nbatch - 2, nbatch - 1):
            for h in range(_NH):
                s_copy(i, h, i % _R).wait()

        @pl.when(has_pad)
        def _():
            for h in range(_NH):
                pltpu.make_async_copy(
                    out_hbm.at[prow, pl.ds(h * h_w, h_w)],
                    rows[h][ps], gsem[h][ps]).wait()
                dst = o2_hbm.at[prow, pl.ds(h * h_w, h_w)]
                pltpu.make_async_copy(rows[h][ps], dst, ssem[h][ps]).start()
            for h in range(_NH):
                dst = o2_hbm.at[prow, pl.ds(h * h_w, h_w)]
                pltpu.make_async_copy(rows[h][ps], dst, ssem[h][ps]).wait()

    return k(inp, out, offs16, spl16)


def kernel(inp, out, in_splits_offsets, out_splits_offsets):
    iso = in_splits_offsets.astype(jnp.int32)
    oso = out_splits_offsets.astype(jnp.int32)
    offs16 = jnp.concatenate([iso[1], oso[1]])
    spl16 = jnp.concatenate([iso[0], jnp.zeros((8,), jnp.int32)])
    return _combine(inp, out, offs16, spl16,
                    in_len=inp.shape[0], out_len=out.shape[0],
                    d=inp.shape[1])


# speedup vs baseline: 1.0371x; 1.0201x over previous
import functools

import jax
import jax.numpy as jnp
from jax import lax
from jax.experimental import pallas as pl
from jax.experimental.pallas import tpu as pltpu
from jax.experimental.pallas import tpu_sc as plsc

_NC = 2
_NS = 16
_NW = _NC * _NS
_L = 16
_B = 16
_R = 3
_NH = 2
_NCHUNK = 8


def _combine(inp, out, offs16, spl16, *, in_len, out_len, d):
    rows_per_w = in_len // _NW
    nbatch = rows_per_w // _B
    h_w = d // _NH
    assert nbatch % _R == 2 and nbatch >= 8 and d % (_NH * 128) == 0

    mesh = plsc.VectorSubcoreMesh(core_axis_name="c", subcore_axis_name="s")

    @functools.partial(
        pl.kernel,
        mesh=mesh,
        out_type=jax.ShapeDtypeStruct((out_len, d), jnp.float32),
        scratch_types=[
            pltpu.VMEM((_L,), jnp.int32),
            pltpu.VMEM((_L,), jnp.int32),
        ]
        + [pltpu.VMEM((_B, h_w), jnp.float32) for _ in range(_NH * _R)]
        + [pltpu.SemaphoreType.DMA for _ in range(2 * _NH * _R)],
    )
    def k(inp_hbm, out_hbm, offs_hbm, spl_hbm, o2_hbm, offs_v, spl_v, *ring):
        wid = lax.axis_index("s") * _NC + lax.axis_index("c")
        iota = lax.iota(jnp.int32, _L)
        nslot = _NH * _R
        rows = [ring[h * _R:(h + 1) * _R] for h in range(_NH)]
        gsem = [ring[nslot + h * _R:nslot + (h + 1) * _R] for h in range(_NH)]
        ssem = [ring[2 * nslot + h * _R:2 * nslot + (h + 1) * _R]
                for h in range(_NH)]

        base = wid * rows_per_w

        def g_slice(i, h):
            r0 = pl.multiple_of(base + i * _B, _B)
            return inp_hbm.at[pl.ds(r0, _B), pl.ds(h * h_w, h_w)]

        for h in range(_NH):
            pltpu.make_async_copy(g_slice(0, h), rows[h][0], gsem[h][0]).start()
        pltpu.make_async_copy(offs_hbm, offs_v, ssem[0][_R - 1]).start()
        pltpu.make_async_copy(spl_hbm, spl_v, ssem[1 % _NH][_R - 1]).start()
        pltpu.make_async_copy(offs_hbm, offs_v, ssem[0][_R - 1]).wait()
        pltpu.make_async_copy(spl_hbm, spl_v, ssem[1 % _NH][_R - 1]).wait()
        offs = offs_v[...]
        spl = spl_v[...]
        in_off = [offs[c] for c in range(_NCHUNK)]
        out_off = [offs[c + 8] for c in range(_NCHUNK)]
        splits = [spl[c] for c in range(_NCHUNK)]
        shift = [out_off[c] - in_off[c] for c in range(_NCHUNK)]
        end = [out_off[c] + splits[c] for c in range(_NCHUNK)]
        gpre, acc = [], jnp.int32(0)
        for c in range(_NCHUNK):
            gpre.append(acc)
            nxt = out_off[c + 1] if c + 1 < _NCHUNK else jnp.int32(out_len)
            acc = acc + (nxt - end[c])

        def dst_rows(i):
            r = pl.multiple_of(base + i * _B, _B) + iota
            cnt = jnp.zeros((_L,), jnp.int32)
            for c in range(1, _NCHUNK):
                cnt = cnt + jnp.where(r >= in_off[c], 1, 0)
            sh = jnp.zeros((_L,), jnp.int32)
            for c in range(1, _NCHUNK):
                sh = jnp.where(cnt == c, shift[c], sh)
            return r + sh

        def g_copy(i, h, s):
            return pltpu.make_async_copy(g_slice(i, h), rows[h][s], gsem[h][s])

        def s_copy(i, h, s):
            dst = o2_hbm.at[dst_rows(i), pl.ds(h * h_w, h_w)]
            return pltpu.make_async_copy(rows[h][s], dst, ssem[h][s])

        def step(i, *, head, tail):
            s_cur = i % _R
            s_old = (i + 1) % _R
            for h in range(_NH):
                if not head:
                    s_copy(i - 2, h, s_old).wait()
                if not tail:
                    g_copy(i + 1, h, s_old).start()
            for h in range(_NH):
                g_copy(i, h, s_cur).wait()
                s_copy(i, h, s_cur).start()

        step(0, head=True, tail=False)
        step(1, head=True, tail=False)
        step(2, head=False, tail=False)

        def group(sg, carry):
            for kk in range(_R):
                i = _R * sg + kk
                s_cur = kk
                s_old = (kk + 1) % _R
                for h in range(_NH):
                    s_copy(i - 2, h, s_old).wait()
                    g_copy(i + 1, h, s_old).start()
                for h in range(_NH):
                    g_copy(i, h, s_cur).wait()
                    s_copy(i, h, s_cur).start()
            return carry

        lax.fori_loop(1, nbatch // _R, group, 0)

        step(nbatch - 2, head=False, tail=False)
        step(nbatch - 1, head=False, tail=True)

        has_pad = wid < (out_len - in_len) // _L
        p = wid * _L + iota
        cnt = jnp.zeros((_L,), jnp.int32)
        for c in range(1, _NCHUNK):
            cnt = cnt + jnp.where(p >= gpre[c], 1, 0)
        rb = jnp.full((_L,), end[0] - gpre[0], jnp.int32)
        for c in range(1, _NCHUNK):
            rb = jnp.where(cnt == c, end[c] - gpre[c], rb)
        prow = rb + p
        ps = (nbatch - 3) % _R

        @pl.when(has_pad)
        def _():
            for h in range(_NH):
                src = out_hbm.at[prow, pl.ds(h * h_w, h_w)]
                pltpu.make_async_copy(src, rows[h][ps], gsem[h][ps]).start()

        for i in (nbatch - 2, nbatch - 1):
            for h in range(_NH):
                s_copy(i, h, i % _R).wait()

        @pl.when(has_pad)
        def _():
            for h in range(_NH):
                pltpu.make_async_copy(
                    out_hbm.at[prow, pl.ds(h * h_w, h_w)],
                    rows[h][ps], gsem[h][ps]).wait()
                dst = o2_hbm.at[prow, pl.ds(h * h_w, h_w)]
                pltpu.make_async_copy(rows[h][ps], dst, ssem[h][ps]).start()
            for h in range(_NH):
                dst = o2_hbm.at[prow, pl.ds(h * h_w, h_w)]
                pltpu.make_async_copy(rows[h][ps], dst, ssem[h][ps]).wait()

    return k(inp, out, offs16, spl16)


def kernel(inp, out, in_splits_offsets, out_splits_offsets):
    iso = in_splits_offsets.astype(jnp.int32)
    oso = out_splits_offsets.astype(jnp.int32)
    offs16 = jnp.concatenate([iso[1], oso[1]])
    spl16 = jnp.concatenate([iso[0], jnp.zeros((8,), jnp.int32)])
    return _combine(inp, out, offs16, spl16,
                    in_len=inp.shape[0], out_len=out.shape[0],
                    d=inp.shape[1])
